# baseline (speedup 1.0000x reference)
import functools

import jax
import jax.numpy as jnp
from jax.experimental import pallas as pl

N_EXPERTS = 64
TOP_K = 2
BLOCK_TOKENS = 2048


def _router_block(x_ref, w_ref, probs_ref, idx_ref, wts_ref):
    x = x_ref[...]
    w = w_ref[...]
    lt = jax.lax.dot_general(
        w, x, (((1,), (1,)), ((), ())), preferred_element_type=jnp.float32
    )
    iota = jax.lax.broadcasted_iota(jnp.int32, lt.shape, 0)
    m = jnp.max(lt, axis=0, keepdims=True)
    i1 = jnp.min(jnp.where(lt == m, iota, N_EXPERTS), axis=0, keepdims=True)
    masked = jnp.where(iota == i1, -jnp.inf, lt)
    m2 = jnp.max(masked, axis=0, keepdims=True)
    i2 = jnp.min(jnp.where(masked == m2, iota, N_EXPERTS), axis=0, keepdims=True)

    e = jnp.exp(lt - m)
    s = jnp.sum(e, axis=0, keepdims=True)
    probs_ref[...] = e / s
    e2 = jnp.exp(m2 - m)
    rd = 1.0 / (1.0 + e2 + 1e-9 * s)
    idx_ref[...] = jnp.concatenate([i1, i2], axis=0)
    wts_ref[...] = jnp.concatenate([rd, e2 * rd], axis=0)


@functools.partial(jax.jit, static_argnames=("interpret",))
def kernel(x, W, interpret=False):
    if x.ndim == 3:
        x = x.reshape(-1, x.shape[-1])
    n_tokens, d_model = x.shape
    n_blocks = n_tokens // BLOCK_TOKENS
    probs2, idx_t, wts_t = pl.pallas_call(
        _router_block,
        grid=(n_blocks,),
        in_specs=[
            pl.BlockSpec((BLOCK_TOKENS, d_model), lambda i: (i, 0)),
            pl.BlockSpec((N_EXPERTS, d_model), lambda i: (0, 0)),
        ],
        out_specs=[
            pl.BlockSpec((N_EXPERTS, BLOCK_TOKENS), lambda i: (0, i)),
            pl.BlockSpec((TOP_K, BLOCK_TOKENS), lambda i: (0, i)),
            pl.BlockSpec((TOP_K, BLOCK_TOKENS), lambda i: (0, i)),
        ],
        out_shape=[
            jax.ShapeDtypeStruct((N_EXPERTS, n_tokens), jnp.float32),
            jax.ShapeDtypeStruct((TOP_K, n_tokens), jnp.int32),
            jax.ShapeDtypeStruct((TOP_K, n_tokens), jnp.float32),
        ],
        interpret=interpret,
    )(x, W)
    return (probs2.T, idx_t.T, wts_t.T)


# speedup vs baseline: 1.0747x; 1.0747x over previous
import jax
import jax.numpy as jnp
from jax.experimental import pallas as pl

N_EXPERTS = 64
TOP_K = 2
BLOCK_TOKENS = 4096


def _router_block(x_ref, w_ref, probs_ref, idx_ref, wts_ref):
    x = x_ref[...]
    w = w_ref[...]
    lt = jax.lax.dot_general(
        w, x, (((1,), (1,)), ((), ())), preferred_element_type=jnp.float32
    )
    iota = jax.lax.broadcasted_iota(jnp.int32, lt.shape, 0)
    m = jnp.max(lt, axis=0, keepdims=True)
    i1 = jnp.min(jnp.where(lt == m, iota, N_EXPERTS), axis=0, keepdims=True)
    masked = jnp.where(iota == i1, -jnp.inf, lt)
    m2 = jnp.max(masked, axis=0, keepdims=True)
    i2 = jnp.min(jnp.where(masked == m2, iota, N_EXPERTS), axis=0, keepdims=True)

    e = jnp.exp(lt - m)
    s = jnp.sum(e, axis=0, keepdims=True)
    probs_ref[...] = e / s
    e2 = jnp.exp(m2 - m)
    rd = 1.0 / (1.0 + e2 + 1e-9 * s)
    idx_ref[...] = jnp.concatenate([i1, i2], axis=0)
    wts_ref[...] = jnp.concatenate([rd, e2 * rd], axis=0)


@jax.jit
def kernel(x, W):
    if x.ndim == 3:
        x = x.reshape(-1, x.shape[-1])
    n_tokens, d_model = x.shape
    n_blocks = n_tokens // BLOCK_TOKENS
    probs2, idx_t, wts_t = pl.pallas_call(
        _router_block,
        grid=(n_blocks,),
        in_specs=[
            pl.BlockSpec((BLOCK_TOKENS, d_model), lambda i: (i, 0)),
            pl.BlockSpec((N_EXPERTS, d_model), lambda i: (0, 0)),
        ],
        out_specs=[
            pl.BlockSpec((N_EXPERTS, BLOCK_TOKENS), lambda i: (0, i)),
            pl.BlockSpec((TOP_K, BLOCK_TOKENS), lambda i: (0, i)),
            pl.BlockSpec((TOP_K, BLOCK_TOKENS), lambda i: (0, i)),
        ],
        out_shape=[
            jax.ShapeDtypeStruct((N_EXPERTS, n_tokens), jnp.float32),
            jax.ShapeDtypeStruct((TOP_K, n_tokens), jnp.int32),
            jax.ShapeDtypeStruct((TOP_K, n_tokens), jnp.float32),
        ],
    )(x, W)
    return (probs2.T, idx_t.T, wts_t.T)
